# Initial kernel scaffold; baseline (speedup 1.0000x reference)
import jax
import jax.numpy as jnp
from jax.experimental import pallas as pl


def kernel(x, edge_index, W1l, b1, W1r, W2l, b2, W2r, Wlin, blin):
    raise NotImplementedError("write your pallas kernel here")


# speedup vs baseline: 2.6395x; 2.6395x over previous
import functools

import jax
import jax.numpy as jnp
from jax import lax
from jax.experimental import pallas as pl
from jax.experimental.pallas import tpu as pltpu
from jax.experimental.pallas import tpu_sc as plsc

NC = 2
NS = 16
NT = NC * NS
R = 320
TBL = R + 1
CH = 16
CHS = 4
BLK = 1024
RINGR = 16
SBR = 64


_SC_MESH = dict(core_axis_name="c", subcore_axis_name="s", num_cores=NC,
                num_subcores=NS)
_SC_PARAMS = pltpu.CompilerParams(needs_layout_passes=False)


def _make_compact(e_rows, cap_r):
  nblk = e_rows // SBR
  out_type = (jax.ShapeDtypeStruct((NT * cap_r, 128), jnp.int32),
              jax.ShapeDtypeStruct((NT * 16,), jnp.int32))
  scratch = [
      pltpu.VMEM((2, SBR, 128), jnp.int32),
      pltpu.VMEM((2, SBR, 128), jnp.int32),
      pltpu.VMEM((RINGR, 128), jnp.int32),
      pltpu.VMEM((16,), jnp.int32),
      pltpu.SemaphoreType.DMA,
  ]

  def body(src_hbm, dst_hbm, lists_hbm, meta_hbm, sbuf, dbuf, ring,
           mbuf, lsem):
    c = lax.axis_index("c")
    s = lax.axis_index("s")
    t = c * NS + s
    base = t * R
    slab0 = pl.multiple_of(t * cap_r, 8)
    iota = lax.iota(jnp.int32, 16)
    trash16 = jnp.full((16,), R, jnp.int32)

    pltpu.sync_copy(src_hbm.at[pl.ds(0, SBR)], sbuf.at[0])
    pltpu.sync_copy(dst_hbm.at[pl.ds(0, SBR)], dbuf.at[0])

    def scan_block(blk, carry):
      cur0, fb0 = carry
      bb = blk & 1

      @pl.when(blk + 1 < nblk)
      def _():
        nb0 = pl.multiple_of((blk + 1) * SBR, 8)
        pltpu.async_copy(src_hbm.at[pl.ds(nb0, SBR)], sbuf.at[1 - bb],
                         lsem)
        pltpu.async_copy(dst_hbm.at[pl.ds(nb0, SBR)], dbuf.at[1 - bb],
                         lsem)

      def grp(g, carry2):
        cur, fb = carry2
        r = g >> 3
        off = (g & 7) * 16
        dv = dbuf[bb, r, pl.ds(off, 16)]
        sv = sbuf[bb, r, pl.ds(off, 16)]
        lv = dv - base
        ok = (lv >= 0) & (lv < R)
        oki = ok.astype(jnp.int32)
        pk = sv * 512 + (lv & 511)
        pos = cur + plsc.cumsum(oki) - oki
        plsc.store_scatter(ring, [(pos >> 7) & (RINGR - 1), pos & 127],
                           pk, mask=ok)
        pc = plsc.all_reduce_population_count(ok)
        cur2 = cur + pc[0]
        full = cur2 - fb * BLK >= BLK

        @pl.when(full)
        def _():
          r0 = pl.multiple_of((fb & 1) * (BLK // 128), 8)
          w0 = pl.multiple_of(slab0 + fb * (BLK // 128), 8)
          pltpu.sync_copy(ring.at[pl.ds(r0, BLK // 128)],
                          lists_hbm.at[pl.ds(w0, BLK // 128)])

        return cur2, jnp.where(full, fb + 1, fb)

      cur0, fb0 = lax.fori_loop(0, SBR * 8, grp, (cur0, fb0))

      @pl.when(blk + 1 < nblk)
      def _():
        pltpu.make_async_copy(src_hbm.at[pl.ds(0, SBR)], sbuf.at[1 - bb],
                              lsem).wait()
        pltpu.make_async_copy(dst_hbm.at[pl.ds(0, SBR)], dbuf.at[1 - bb],
                              lsem).wait()
      return cur0, fb0

    cur, fb = lax.fori_loop(0, nblk, scan_block,
                            (jnp.int32(0), jnp.int32(0)))

    pad_end = (cur + CH - 1) & -CH

    def padfill(j, _):
      p = cur + iota + j * 16
      okp = p < pad_end
      plsc.store_scatter(ring, [(p >> 7) & (RINGR - 1), p & 127],
                         trash16, mask=okp)
      return 0

    lax.fori_loop(0, CH // 16, padfill, 0)

    full2 = pad_end - fb * BLK >= BLK

    @pl.when(full2)
    def _():
      r0 = pl.multiple_of((fb & 1) * (BLK // 128), 8)
      w0 = pl.multiple_of(slab0 + fb * (BLK // 128), 8)
      pltpu.sync_copy(ring.at[pl.ds(r0, BLK // 128)],
                      lists_hbm.at[pl.ds(w0, BLK // 128)])

    fb = jnp.where(full2, fb + 1, fb)
    r0 = pl.multiple_of((fb & 1) * (BLK // 128), 8)
    w0 = pl.multiple_of(slab0 + fb * (BLK // 128), 8)
    pltpu.sync_copy(ring.at[pl.ds(r0, BLK // 128)],
                    lists_hbm.at[pl.ds(w0, BLK // 128)])
    mbuf[pl.ds(0, 16)] = jnp.zeros((16,), jnp.int32) + (pad_end >> CHS)
    m0 = pl.multiple_of(t * 16, 8)
    pltpu.sync_copy(mbuf, meta_hbm.at[pl.ds(m0, 16)])

  return pl.kernel(body, out_type=out_type,
                   mesh=plsc.VectorSubcoreMesh(**_SC_MESH),
                   scratch_types=scratch, compiler_params=_SC_PARAMS)


def _make_aggregate(n_pad, d, cap_r):
  out_type = jax.ShapeDtypeStruct((n_pad, d), jnp.float32)
  scratch = [
      pltpu.VMEM((TBL, d), jnp.float32),
      pltpu.VMEM((2, CH, d), jnp.float32),
      pltpu.VMEM((8, 128), jnp.int32),
      pltpu.VMEM((4, 16), jnp.int32),
      pltpu.VMEM((16,), jnp.int32),
      pltpu.SemaphoreType.DMA,
  ]
  def body(x_hbm, lists_hbm, meta_hbm, zrow_hbm, agg_hbm,
           table, rows_v, pblk, gidx, mbuf, gsem):
    c = lax.axis_index("c")
    s = lax.axis_index("s")
    t = c * NS + s
    slab0 = pl.multiple_of(t * cap_r, 8)

    m0 = pl.multiple_of(t * 16, 8)
    pltpu.sync_copy(meta_hbm.at[pl.ds(m0, 16)], mbuf)
    nch = mbuf[pl.ds(0, 16)][0]
    pltpu.sync_copy(zrow_hbm, table.at[pl.ds(0, R)])

    def block(blk, _):
      b0 = pl.multiple_of(slab0 + blk * 8, 8)
      pltpu.sync_copy(lists_hbm.at[pl.ds(b0, 8)], pblk)

      n_here = jnp.minimum(BLK // CH, nch - blk * (BLK // CH))

      def stage(ii):
        v = pblk[ii >> 3, pl.ds((ii & 7) * 16, 16)]
        gidx[ii & 3, pl.ds(0, 16)] = lax.shift_right_logical(v, 9)
        pltpu.async_copy(x_hbm.at[gidx.at[ii & 3]], rows_v.at[ii & 1],
                         gsem)

      @pl.when(n_here >= 1)
      def _():
        stage(jnp.int32(0))

      @pl.when(n_here >= 2)
      def _():
        stage(jnp.int32(1))

      def chunk(cc, _):
        b = cc & 1
        pltpu.make_async_copy(x_hbm.at[gidx.at[0]], rows_v.at[b],
                              gsem).wait()
        lvv = pblk[cc >> 3, pl.ds((cc & 7) * 16, 16)] & 511
        for j in range(16):
          r = lvv[j]
          for cg in range(d // 16):
            plsc.addupdate(table.at[r, pl.ds(cg * 16, 16)],
                           rows_v[b, j, pl.ds(cg * 16, 16)])

        @pl.when(cc + 2 < n_here)
        def _():
          stage(cc + 2)

        return 0

      lax.fori_loop(0, n_here, chunk, 0)
      return 0

    lax.fori_loop(0, (nch + BLK // CH - 1) // (BLK // CH), block, 0)

    w0 = pl.multiple_of(t * R, 8)
    pltpu.sync_copy(table.at[pl.ds(0, R)], agg_hbm.at[pl.ds(w0, R)])

  return pl.kernel(body, out_type=out_type,
                   mesh=plsc.VectorSubcoreMesh(**_SC_MESH),
                   scratch_types=scratch, compiler_params=_SC_PARAMS)


def _make_counts(n_pad, cap_r):
  out_type = jax.ShapeDtypeStruct((n_pad, 16), jnp.float32)
  scratch = [
      pltpu.VMEM((TBL, 16), jnp.float32),
      pltpu.VMEM((8, 128), jnp.int32),
      pltpu.VMEM((16,), jnp.int32),
  ]

  def body(lists_hbm, meta_hbm, z16_hbm, cnt_hbm, cntt, pblk, mbuf):
    ones16 = jnp.ones((16,), jnp.float32)
    c = lax.axis_index("c")
    s = lax.axis_index("s")
    t = c * NS + s
    slab0 = pl.multiple_of(t * cap_r, 8)
    m0 = pl.multiple_of(t * 16, 8)
    pltpu.sync_copy(meta_hbm.at[pl.ds(m0, 16)], mbuf)
    nch = mbuf[pl.ds(0, 16)][0]
    pltpu.sync_copy(z16_hbm, cntt.at[pl.ds(0, R)])

    def block(blk, _):
      b0 = pl.multiple_of(slab0 + blk * 8, 8)
      pltpu.sync_copy(lists_hbm.at[pl.ds(b0, 8)], pblk)
      n_here = jnp.minimum(BLK // CH, nch - blk * (BLK // CH))

      def chunk(cc, _):
        lvv = pblk[cc >> 3, pl.ds((cc & 7) * 16, 16)] & 511
        for j in range(16):
          plsc.addupdate(cntt.at[lvv[j], pl.ds(0, 16)], ones16)
        return 0

      lax.fori_loop(0, n_here, chunk, 0)
      return 0

    lax.fori_loop(0, (nch + BLK // CH - 1) // (BLK // CH), block, 0)
    w0 = pl.multiple_of(t * R, 8)
    pltpu.sync_copy(cntt.at[pl.ds(0, R)], cnt_hbm.at[pl.ds(w0, R)])

  return pl.kernel(body, out_type=out_type,
                   mesh=plsc.VectorSubcoreMesh(**_SC_MESH),
                   scratch_types=scratch, compiler_params=_SC_PARAMS)


def _dotT(a, w):
  return lax.dot_general(a, w, (((1,), (1,)), ((), ())),
                         preferred_element_type=jnp.float32)


def _make_tc_layer(n_pad, d, bm, final):
  def body(*refs):
    if final:
      (agg_ref, cnt_ref, x_ref, wl_ref, b_ref, wr_ref, wlin_ref, blin_ref,
       o_ref) = refs
    else:
      agg_ref, cnt_ref, x_ref, wl_ref, b_ref, wr_ref, o_ref = refs
    inv = 1.0 / jnp.maximum(cnt_ref[:, 0:1], 1.0)
    acc = _dotT(agg_ref[...], wl_ref[...]) * inv
    acc = acc + b_ref[...] + _dotT(x_ref[...], wr_ref[...])
    h = jnp.maximum(acc, 0.0)
    if final:
      o_ref[...] = _dotT(h, wlin_ref[...]) + blin_ref[...]
    else:
      o_ref[...] = h

  row_spec = pl.BlockSpec((bm, d), lambda i: (i, 0))
  w_spec = pl.BlockSpec((d, d), lambda i: (0, 0))
  b_spec = pl.BlockSpec((1, d), lambda i: (0, 0))
  in_specs = [row_spec, pl.BlockSpec((bm, 16), lambda i: (i, 0)), row_spec,
              w_spec, b_spec, w_spec]
  if final:
    in_specs += [w_spec, b_spec]
  return pl.pallas_call(
      body,
      grid=(n_pad // bm,),
      in_specs=in_specs,
      out_specs=row_spec,
      out_shape=jax.ShapeDtypeStruct((n_pad, d), jnp.float32),
  )


@jax.jit
def kernel(x, edge_index, W1l, b1, W1r, W2l, b2, W2r, Wlin, blin):
  x = x.astype(jnp.float32)
  n, d = x.shape
  e = edge_index.shape[1]

  n_pad = NT * R
  xp = jnp.pad(x, ((0, n_pad - n), (0, 0)))

  grp = SBR * 128
  e_pad = -(-e // grp) * grp
  e_rows = e_pad // 128
  src = jnp.concatenate(
      [edge_index[0], jnp.zeros((e_pad - e,), jnp.int32)]).reshape(-1, 128)
  dst = jnp.concatenate(
      [edge_index[1], jnp.full((e_pad - e,), -1, jnp.int32)]).reshape(-1, 128)
  cap_r = (-(-(e_pad + 128) // BLK) * (BLK // 128) + 15) // 8 * 8

  zrow = jnp.zeros((R, d), jnp.float32)
  z16 = jnp.zeros((R, 16), jnp.float32)

  compact = _make_compact(e_rows, cap_r)
  agg = _make_aggregate(n_pad, d, cap_r)
  counts = _make_counts(n_pad, cap_r)
  bm = 256
  tc1 = _make_tc_layer(n_pad, d, bm, final=False)
  tc2 = _make_tc_layer(n_pad, d, bm, final=True)

  lists, meta = compact(src, dst)
  cnt = counts(lists, meta, z16)
  agg1 = agg(xp, lists, meta, zrow)
  h1 = tc1(agg1, cnt, xp, W1l, b1.reshape(1, -1), W1r)
  agg2 = agg(h1, lists, meta, zrow)
  out = tc2(agg2, cnt, h1, W2l, b2.reshape(1, -1), W2r, Wlin,
            blin.reshape(1, -1))
  return out[:n]


# speedup vs baseline: 2.6515x; 1.0046x over previous
import functools

import jax
import jax.numpy as jnp
from jax import lax
from jax.experimental import pallas as pl
from jax.experimental.pallas import tpu as pltpu
from jax.experimental.pallas import tpu_sc as plsc

NC = 2
NS = 16
NT = NC * NS
R = 320
TBL = R + 1
CH = 16
CHS = 4
BLK = 1024
RINGR = 16
SBR = 64


_SC_MESH = dict(core_axis_name="c", subcore_axis_name="s", num_cores=NC,
                num_subcores=NS)
_SC_PARAMS = pltpu.CompilerParams(needs_layout_passes=False)


def _make_compact(e_rows, cap_r):
  nblk = e_rows // SBR
  out_type = (jax.ShapeDtypeStruct((NT * cap_r, 128), jnp.int32),
              jax.ShapeDtypeStruct((NT * 16,), jnp.int32))
  scratch = [
      pltpu.VMEM((2, SBR, 128), jnp.int32),
      pltpu.VMEM((2, SBR, 128), jnp.int32),
      pltpu.VMEM((RINGR, 128), jnp.int32),
      pltpu.VMEM((16,), jnp.int32),
      pltpu.SemaphoreType.DMA,
  ]

  def body(src_hbm, dst_hbm, lists_hbm, meta_hbm, sbuf, dbuf, ring,
           mbuf, lsem):
    c = lax.axis_index("c")
    s = lax.axis_index("s")
    t = c * NS + s
    base = t * R
    slab0 = pl.multiple_of(t * cap_r, 8)
    iota = lax.iota(jnp.int32, 16)
    trash16 = jnp.full((16,), R, jnp.int32)

    pltpu.sync_copy(src_hbm.at[pl.ds(0, SBR)], sbuf.at[0])
    pltpu.sync_copy(dst_hbm.at[pl.ds(0, SBR)], dbuf.at[0])

    def scan_block(blk, carry):
      cur0, fb0 = carry
      bb = blk & 1

      @pl.when(blk + 1 < nblk)
      def _():
        nb0 = pl.multiple_of((blk + 1) * SBR, 8)
        pltpu.async_copy(src_hbm.at[pl.ds(nb0, SBR)], sbuf.at[1 - bb],
                         lsem)
        pltpu.async_copy(dst_hbm.at[pl.ds(nb0, SBR)], dbuf.at[1 - bb],
                         lsem)

      def grp(g, carry2):
        cur, fb = carry2
        r = g >> 3
        off = (g & 7) * 16
        dv = dbuf[bb, r, pl.ds(off, 16)]
        sv = sbuf[bb, r, pl.ds(off, 16)]
        lv = dv - base
        ok = (lv >= 0) & (lv < R)
        oki = ok.astype(jnp.int32)
        pk = sv * 512 + (lv & 511)
        pos = cur + plsc.cumsum(oki) - oki
        plsc.store_scatter(ring, [(pos >> 7) & (RINGR - 1), pos & 127],
                           pk, mask=ok)
        pc = plsc.all_reduce_population_count(ok)
        cur2 = cur + pc[0]
        full = cur2 - fb * BLK >= BLK

        @pl.when(full)
        def _():
          r0 = pl.multiple_of((fb & 1) * (BLK // 128), 8)
          w0 = pl.multiple_of(slab0 + fb * (BLK // 128), 8)
          pltpu.sync_copy(ring.at[pl.ds(r0, BLK // 128)],
                          lists_hbm.at[pl.ds(w0, BLK // 128)])

        return cur2, jnp.where(full, fb + 1, fb)

      cur0, fb0 = lax.fori_loop(0, SBR * 8, grp, (cur0, fb0))

      @pl.when(blk + 1 < nblk)
      def _():
        pltpu.make_async_copy(src_hbm.at[pl.ds(0, SBR)], sbuf.at[1 - bb],
                              lsem).wait()
        pltpu.make_async_copy(dst_hbm.at[pl.ds(0, SBR)], dbuf.at[1 - bb],
                              lsem).wait()
      return cur0, fb0

    cur, fb = lax.fori_loop(0, nblk, scan_block,
                            (jnp.int32(0), jnp.int32(0)))

    pad_end = (cur + CH - 1) & -CH

    def padfill(j, _):
      p = cur + iota + j * 16
      okp = p < pad_end
      plsc.store_scatter(ring, [(p >> 7) & (RINGR - 1), p & 127],
                         trash16, mask=okp)
      return 0

    lax.fori_loop(0, CH // 16, padfill, 0)

    full2 = pad_end - fb * BLK >= BLK

    @pl.when(full2)
    def _():
      r0 = pl.multiple_of((fb & 1) * (BLK // 128), 8)
      w0 = pl.multiple_of(slab0 + fb * (BLK // 128), 8)
      pltpu.sync_copy(ring.at[pl.ds(r0, BLK // 128)],
                      lists_hbm.at[pl.ds(w0, BLK // 128)])

    fb = jnp.where(full2, fb + 1, fb)
    r0 = pl.multiple_of((fb & 1) * (BLK // 128), 8)
    w0 = pl.multiple_of(slab0 + fb * (BLK // 128), 8)
    pltpu.sync_copy(ring.at[pl.ds(r0, BLK // 128)],
                    lists_hbm.at[pl.ds(w0, BLK // 128)])
    mbuf[pl.ds(0, 16)] = jnp.zeros((16,), jnp.int32) + (pad_end >> CHS)
    m0 = pl.multiple_of(t * 16, 8)
    pltpu.sync_copy(mbuf, meta_hbm.at[pl.ds(m0, 16)])

  return pl.kernel(body, out_type=out_type,
                   mesh=plsc.VectorSubcoreMesh(**_SC_MESH),
                   scratch_types=scratch, compiler_params=_SC_PARAMS)


def _make_aggregate(n_pad, d, cap_r):
  out_type = jax.ShapeDtypeStruct((n_pad, d), jnp.float32)
  scratch = [
      pltpu.VMEM((TBL, d), jnp.float32),
      pltpu.VMEM((4, CH, d), jnp.float32),
      pltpu.VMEM((8, 128), jnp.int32),
      pltpu.VMEM((8, 16), jnp.int32),
      pltpu.VMEM((16,), jnp.int32),
      pltpu.SemaphoreType.DMA,
  ]
  def body(x_hbm, lists_hbm, meta_hbm, zrow_hbm, agg_hbm,
           table, rows_v, pblk, gidx, mbuf, gsem):
    c = lax.axis_index("c")
    s = lax.axis_index("s")
    t = c * NS + s
    slab0 = pl.multiple_of(t * cap_r, 8)

    m0 = pl.multiple_of(t * 16, 8)
    pltpu.sync_copy(meta_hbm.at[pl.ds(m0, 16)], mbuf)
    nch = mbuf[pl.ds(0, 16)][0]
    pltpu.sync_copy(zrow_hbm, table.at[pl.ds(0, R)])

    def block(blk, _):
      b0 = pl.multiple_of(slab0 + blk * 8, 8)
      pltpu.sync_copy(lists_hbm.at[pl.ds(b0, 8)], pblk)

      n_here = jnp.minimum(BLK // CH, nch - blk * (BLK // CH))

      def stage(ii):
        v = pblk[ii >> 3, pl.ds((ii & 7) * 16, 16)]
        gidx[ii & 7, pl.ds(0, 16)] = lax.shift_right_logical(v, 9)
        pltpu.async_copy(x_hbm.at[gidx.at[ii & 7]], rows_v.at[ii & 3],
                         gsem)

      for q in range(4):
        @pl.when(n_here >= q + 1)
        def _(q=q):
          stage(jnp.int32(q))

      def chunk(cc, _):
        b = cc & 3
        pltpu.make_async_copy(x_hbm.at[gidx.at[0]], rows_v.at[b],
                              gsem).wait()
        lvv = pblk[cc >> 3, pl.ds((cc & 7) * 16, 16)] & 511
        for j in range(16):
          r = lvv[j]
          for cg in range(d // 16):
            plsc.addupdate(table.at[r, pl.ds(cg * 16, 16)],
                           rows_v[b, j, pl.ds(cg * 16, 16)])

        @pl.when(cc + 4 < n_here)
        def _():
          stage(cc + 4)

        return 0

      lax.fori_loop(0, n_here, chunk, 0)
      return 0

    lax.fori_loop(0, (nch + BLK // CH - 1) // (BLK // CH), block, 0)

    w0 = pl.multiple_of(t * R, 8)
    pltpu.sync_copy(table.at[pl.ds(0, R)], agg_hbm.at[pl.ds(w0, R)])

  return pl.kernel(body, out_type=out_type,
                   mesh=plsc.VectorSubcoreMesh(**_SC_MESH),
                   scratch_types=scratch, compiler_params=_SC_PARAMS)


def _make_counts(n_pad, cap_r):
  out_type = jax.ShapeDtypeStruct((n_pad, 16), jnp.float32)
  scratch = [
      pltpu.VMEM((TBL, 16), jnp.float32),
      pltpu.VMEM((8, 128), jnp.int32),
      pltpu.VMEM((16,), jnp.int32),
  ]

  def body(lists_hbm, meta_hbm, z16_hbm, cnt_hbm, cntt, pblk, mbuf):
    ones16 = jnp.ones((16,), jnp.float32)
    c = lax.axis_index("c")
    s = lax.axis_index("s")
    t = c * NS + s
    slab0 = pl.multiple_of(t * cap_r, 8)
    m0 = pl.multiple_of(t * 16, 8)
    pltpu.sync_copy(meta_hbm.at[pl.ds(m0, 16)], mbuf)
    nch = mbuf[pl.ds(0, 16)][0]
    pltpu.sync_copy(z16_hbm, cntt.at[pl.ds(0, R)])

    def block(blk, _):
      b0 = pl.multiple_of(slab0 + blk * 8, 8)
      pltpu.sync_copy(lists_hbm.at[pl.ds(b0, 8)], pblk)
      n_here = jnp.minimum(BLK // CH, nch - blk * (BLK // CH))

      def chunk(cc, _):
        lvv = pblk[cc >> 3, pl.ds((cc & 7) * 16, 16)] & 511
        for j in range(16):
          plsc.addupdate(cntt.at[lvv[j], pl.ds(0, 16)], ones16)
        return 0

      lax.fori_loop(0, n_here, chunk, 0)
      return 0

    lax.fori_loop(0, (nch + BLK // CH - 1) // (BLK // CH), block, 0)
    w0 = pl.multiple_of(t * R, 8)
    pltpu.sync_copy(cntt.at[pl.ds(0, R)], cnt_hbm.at[pl.ds(w0, R)])

  return pl.kernel(body, out_type=out_type,
                   mesh=plsc.VectorSubcoreMesh(**_SC_MESH),
                   scratch_types=scratch, compiler_params=_SC_PARAMS)


def _dotT(a, w):
  return lax.dot_general(a, w, (((1,), (1,)), ((), ())),
                         preferred_element_type=jnp.float32)


def _make_tc_layer(n_pad, d, bm, final):
  def body(*refs):
    if final:
      (agg_ref, cnt_ref, x_ref, wl_ref, b_ref, wr_ref, wlin_ref, blin_ref,
       o_ref) = refs
    else:
      agg_ref, cnt_ref, x_ref, wl_ref, b_ref, wr_ref, o_ref = refs
    inv = 1.0 / jnp.maximum(cnt_ref[:, 0:1], 1.0)
    acc = _dotT(agg_ref[...], wl_ref[...]) * inv
    acc = acc + b_ref[...] + _dotT(x_ref[...], wr_ref[...])
    h = jnp.maximum(acc, 0.0)
    if final:
      o_ref[...] = _dotT(h, wlin_ref[...]) + blin_ref[...]
    else:
      o_ref[...] = h

  row_spec = pl.BlockSpec((bm, d), lambda i: (i, 0))
  w_spec = pl.BlockSpec((d, d), lambda i: (0, 0))
  b_spec = pl.BlockSpec((1, d), lambda i: (0, 0))
  in_specs = [row_spec, pl.BlockSpec((bm, 16), lambda i: (i, 0)), row_spec,
              w_spec, b_spec, w_spec]
  if final:
    in_specs += [w_spec, b_spec]
  return pl.pallas_call(
      body,
      grid=(n_pad // bm,),
      in_specs=in_specs,
      out_specs=row_spec,
      out_shape=jax.ShapeDtypeStruct((n_pad, d), jnp.float32),
  )


@jax.jit
def kernel(x, edge_index, W1l, b1, W1r, W2l, b2, W2r, Wlin, blin):
  x = x.astype(jnp.float32)
  n, d = x.shape
  e = edge_index.shape[1]

  n_pad = NT * R
  xp = jnp.pad(x, ((0, n_pad - n), (0, 0)))

  grp = SBR * 128
  e_pad = -(-e // grp) * grp
  e_rows = e_pad // 128
  src = jnp.concatenate(
      [edge_index[0], jnp.zeros((e_pad - e,), jnp.int32)]).reshape(-1, 128)
  dst = jnp.concatenate(
      [edge_index[1], jnp.full((e_pad - e,), -1, jnp.int32)]).reshape(-1, 128)
  cap_r = (-(-(e_pad + 128) // BLK) * (BLK // 128) + 15) // 8 * 8

  zrow = jnp.zeros((R, d), jnp.float32)
  z16 = jnp.zeros((R, 16), jnp.float32)

  compact = _make_compact(e_rows, cap_r)
  agg = _make_aggregate(n_pad, d, cap_r)
  counts = _make_counts(n_pad, cap_r)
  bm = 256
  tc1 = _make_tc_layer(n_pad, d, bm, final=False)
  tc2 = _make_tc_layer(n_pad, d, bm, final=True)

  lists, meta = compact(src, dst)
  cnt = counts(lists, meta, z16)
  agg1 = agg(xp, lists, meta, zrow)
  h1 = tc1(agg1, cnt, xp, W1l, b1.reshape(1, -1), W1r)
  agg2 = agg(h1, lists, meta, zrow)
  out = tc2(agg2, cnt, h1, W2l, b2.reshape(1, -1), W2r, Wlin,
            blin.reshape(1, -1))
  return out[:n]
